# baseline (speedup 1.0000x reference)
import functools
import math

import jax
import jax.numpy as jnp
from jax import lax
from jax.experimental import pallas as pl
from jax.experimental.pallas import tpu as pltpu
from jax.experimental.pallas import tpu_sc as plsc

_EMB = 64
_SCALE = math.sqrt(_EMB)
_CHUNK = 1024
_IGROUP = 128


@functools.lru_cache(maxsize=None)
def _build(B, V):
    info = plsc.get_sparse_core_info()
    nw = info.num_cores * info.num_subcores
    assert B % (nw * _CHUNK) == 0
    bpw = B // nw
    chunks = bpw // _CHUNK
    g = _CHUNK // _IGROUP
    mesh = plsc.VectorSubcoreMesh(core_axis_name="c", subcore_axis_name="s")

    @functools.partial(
        pl.kernel,
        mesh=mesh,
        compiler_params=pltpu.CompilerParams(use_tc_tiling_on_sc=False),
        out_type=jax.ShapeDtypeStruct((B, _EMB), jnp.float32),
        scratch_types=[
            pltpu.VMEM((g, _IGROUP), jnp.int32),
            pltpu.VMEM((_CHUNK, _EMB), jnp.float32),
            pltpu.SemaphoreType.DMA,
        ],
    )
    def embed(idx_hbm, table_hbm, out_hbm, idx_v, rows_v, gsem):
        wid = lax.axis_index("s") * info.num_cores + lax.axis_index("c")

        def chunk_body(c, carry):
            row0 = (wid * chunks + c) * g
            pltpu.sync_copy(idx_hbm.at[pl.ds(row0, g)], idx_v)
            for j in range(g):
                pltpu.make_async_copy(
                    table_hbm.at[idx_v.at[j]],
                    rows_v.at[pl.ds(j * _IGROUP, _IGROUP)],
                    gsem,
                ).start()
            for j in range(g):
                pltpu.make_async_copy(
                    table_hbm.at[idx_v.at[j]],
                    rows_v.at[pl.ds(j * _IGROUP, _IGROUP)],
                    gsem,
                ).wait()

            def row_body(i, rcarry):
                for k in range(_EMB // 16):
                    sl = pl.ds(k * 16, 16)
                    rows_v[i, sl] = rows_v[i, sl] * _SCALE
                return rcarry

            lax.fori_loop(0, _CHUNK, row_body, 0, unroll=2)
            out0 = (wid * chunks + c) * _CHUNK
            pltpu.sync_copy(rows_v, out_hbm.at[pl.ds(out0, _CHUNK)])
            return carry

        lax.fori_loop(0, chunks, chunk_body, 0)

    return embed


@jax.jit
def kernel(tokens, embedding_weight):
    batch, seq = tokens.shape
    b = batch * seq
    idx2d = tokens.reshape(b // _IGROUP, _IGROUP).astype(jnp.int32)
    out = _build(b, embedding_weight.shape[0])(idx2d, embedding_weight)
    return out.reshape(batch, seq, _EMB)


# speedup vs baseline: 1.1007x; 1.1007x over previous
import functools
import math

import jax
import jax.numpy as jnp
from jax import lax
from jax.experimental import pallas as pl
from jax.experimental.pallas import tpu as pltpu
from jax.experimental.pallas import tpu_sc as plsc

_EMB = 64
_SCALE = math.sqrt(_EMB)
_CHUNK = 640
_IGROUP = 128


@functools.lru_cache(maxsize=None)
def _build(B):
    info = plsc.get_sparse_core_info()
    nw = info.num_cores * info.num_subcores
    assert B % (nw * 2 * _CHUNK) == 0
    bpw = B // nw
    chunks = bpw // _CHUNK
    g = _CHUNK // _IGROUP
    mesh = plsc.VectorSubcoreMesh(core_axis_name="c", subcore_axis_name="s")

    @functools.partial(
        pl.kernel,
        mesh=mesh,
        compiler_params=pltpu.CompilerParams(use_tc_tiling_on_sc=False),
        out_type=jax.ShapeDtypeStruct((B, _EMB), jnp.float32),
        scratch_types=[
            pltpu.VMEM((2, g, _IGROUP), jnp.int32),
            pltpu.VMEM((2, _CHUNK, _EMB), jnp.float32),
            [pltpu.SemaphoreType.DMA] * 2,
            [pltpu.SemaphoreType.DMA] * 2,
            [pltpu.SemaphoreType.DMA] * 2,
        ],
    )
    def embed(idx_hbm, table_hbm, out_hbm, idx_v, rows_v, gsem, ssem, isem):
        wid = lax.axis_index("s") * info.num_cores + lax.axis_index("c")
        base = wid * chunks

        def fire_idx(chunk, p):
            pltpu.make_async_copy(
                idx_hbm.at[pl.ds((base + chunk) * g, g)], idx_v.at[p], isem[p]
            ).start()

        def fire_gathers(p):
            for j in range(g):
                pltpu.make_async_copy(
                    table_hbm.at[idx_v.at[p, j]],
                    rows_v.at[p, pl.ds(j * _IGROUP, _IGROUP)],
                    gsem[p],
                ).start()

        def drain_gathers(p):
            for j in range(g):
                pltpu.make_async_copy(
                    table_hbm.at[idx_v.at[p, j]],
                    rows_v.at[p, pl.ds(j * _IGROUP, _IGROUP)],
                    gsem[p],
                ).wait()

        def scatter(chunk, p, wait):
            cp = pltpu.make_async_copy(
                rows_v.at[p], out_hbm.at[pl.ds((base + chunk) * _CHUNK, _CHUNK)],
                ssem[p],
            )
            cp.wait() if wait else cp.start()

        def step(chunk, p):
            q = 1 - p
            drain_gathers(p)
            @pl.when(jnp.logical_and(chunk > 0, chunk + 1 < chunks))
            def _():
                scatter(chunk - 1, q, wait=True)

            @pl.when(chunk + 1 < chunks)
            def _():
                pltpu.make_async_copy(
                    idx_hbm.at[pl.ds((base + chunk + 1) * g, g)],
                    idx_v.at[q], isem[q],
                ).wait()
                fire_gathers(q)

            @pl.when(chunk + 2 < chunks)
            def _():
                fire_idx(chunk + 2, p)

            @plsc.parallel_loop(0, _CHUNK, unroll=4)
            def _(i):
                for k in range(_EMB // 16):
                    sl = pl.ds(k * 16, 16)
                    rows_v[p, i, sl] = rows_v[p, i, sl] * _SCALE

            scatter(chunk, p, wait=False)

        pltpu.sync_copy(idx_hbm.at[pl.ds(base * g, g)], idx_v.at[0])
        fire_gathers(0)
        fire_idx(1, 1)

        def pair(h, carry):
            step(2 * h, 0)
            step(2 * h + 1, 1)
            return carry

        lax.fori_loop(0, chunks // 2, pair, 0)
        scatter(chunks - 2, 0, wait=True)
        scatter(chunks - 1, 1, wait=True)

    return embed


@jax.jit
def kernel(tokens, embedding_weight):
    batch, seq = tokens.shape
    b = batch * seq
    idx2d = tokens.reshape(b // _IGROUP, _IGROUP).astype(jnp.int32)
    out = _build(b)(idx2d, embedding_weight)
    return out.reshape(batch, seq, _EMB)


# speedup vs baseline: 1.1035x; 1.0025x over previous
import functools
import math

import jax
import jax.numpy as jnp
from jax import lax
from jax.experimental import pallas as pl
from jax.experimental.layout import Format, Layout
from jax.experimental.pallas import tpu as pltpu
from jax.experimental.pallas import tpu_sc as plsc

_EMB = 64
_SCALE = math.sqrt(_EMB)
_CHUNK = 640
_IGROUP = 128


@functools.lru_cache(maxsize=None)
def _build(B):
    info = plsc.get_sparse_core_info()
    nw = info.num_cores * info.num_subcores
    assert B % (nw * 2 * _CHUNK) == 0
    bpw = B // nw
    chunks = bpw // _CHUNK
    g = _CHUNK // _IGROUP
    mesh = plsc.VectorSubcoreMesh(core_axis_name="c", subcore_axis_name="s")

    @functools.partial(
        pl.kernel,
        mesh=mesh,
        compiler_params=pltpu.CompilerParams(use_tc_tiling_on_sc=False),
        out_type=jax.ShapeDtypeStruct((B, _EMB), jnp.float32),
        scratch_types=[
            pltpu.VMEM((2, g, _IGROUP), jnp.int32),
            pltpu.VMEM((2, _CHUNK, _EMB), jnp.float32),
            [pltpu.SemaphoreType.DMA] * 2,
            [pltpu.SemaphoreType.DMA] * 2,
            [pltpu.SemaphoreType.DMA] * 2,
        ],
    )
    def embed(idx_hbm, table_hbm, out_hbm, idx_v, rows_v, gsem, ssem, isem):
        wid = lax.axis_index("s") * info.num_cores + lax.axis_index("c")
        base = wid * chunks

        def fire_idx(chunk, p):
            pltpu.make_async_copy(
                idx_hbm.at[pl.ds((base + chunk) * g, g)], idx_v.at[p], isem[p]
            ).start()

        def fire_gathers(p):
            for j in range(g):
                pltpu.make_async_copy(
                    table_hbm.at[idx_v.at[p, j]],
                    rows_v.at[p, pl.ds(j * _IGROUP, _IGROUP)],
                    gsem[p],
                ).start()

        def drain_gathers(p):
            for j in range(g):
                pltpu.make_async_copy(
                    table_hbm.at[idx_v.at[p, j]],
                    rows_v.at[p, pl.ds(j * _IGROUP, _IGROUP)],
                    gsem[p],
                ).wait()

        def scatter(chunk, p, wait):
            cp = pltpu.make_async_copy(
                rows_v.at[p], out_hbm.at[pl.ds((base + chunk) * _CHUNK, _CHUNK)],
                ssem[p],
            )
            cp.wait() if wait else cp.start()

        def step(chunk, p):
            q = 1 - p
            drain_gathers(p)
            @pl.when(jnp.logical_and(chunk > 0, chunk + 1 < chunks))
            def _():
                scatter(chunk - 1, q, wait=True)

            @pl.when(chunk + 1 < chunks)
            def _():
                pltpu.make_async_copy(
                    idx_hbm.at[pl.ds((base + chunk + 1) * g, g)],
                    idx_v.at[q], isem[q],
                ).wait()
                fire_gathers(q)

            @pl.when(chunk + 2 < chunks)
            def _():
                fire_idx(chunk + 2, p)

            @plsc.parallel_loop(0, _CHUNK, unroll=4)
            def _(i):
                for k in range(_EMB // 16):
                    sl = pl.ds(k * 16, 16)
                    rows_v[p, i, sl] = rows_v[p, i, sl] * _SCALE

            scatter(chunk, p, wait=False)

        pltpu.sync_copy(idx_hbm.at[pl.ds(base * g, g)], idx_v.at[0])
        fire_gathers(0)
        fire_idx(1, 1)

        def pair(h, carry):
            step(2 * h, 0)
            step(2 * h + 1, 1)
            return carry

        lax.fori_loop(0, chunks // 2, pair, 0)
        scatter(chunks - 2, 0, wait=True)
        scatter(chunks - 1, 1, wait=True)

    return embed


def _impl(tokens, embedding_weight):
    batch, seq = tokens.shape
    b = batch * seq
    idx2d = tokens.reshape(b // _IGROUP, _IGROUP).astype(jnp.int32)
    out = _build(b)(idx2d, embedding_weight)
    return out.reshape(batch, seq, _EMB)


@functools.lru_cache(maxsize=None)
def _jitted(device):
    fmt = Format(
        Layout(major_to_minor=(0, 1, 2), tiling=()),
        jax.sharding.SingleDeviceSharding(device),
    )
    return jax.jit(_impl, out_shardings=fmt)


def _default_device():
    try:
        from jax._src.mesh import get_concrete_mesh

        mesh = get_concrete_mesh()
        if mesh is not None and getattr(mesh, "devices", None) is not None:
            dev = mesh.devices.flat[0]
            if dev is not None:
                return dev
    except Exception:
        pass
    try:
        return jax.devices("tpu")[0]
    except Exception:
        return jax.devices()[0]


def kernel(tokens, embedding_weight):
    return _jitted(_default_device())(tokens, embedding_weight)
